# baseline (speedup 1.0000x reference)
import functools

import jax
import jax.numpy as jnp
from jax import lax
from jax.experimental import pallas as pl
from jax.experimental.pallas import tpu as pltpu
from jax.experimental.pallas import tpu_sc as plsc

B, D, P, L = 16384, 768, 100, 8
DK = (L // 2) * D

TC_BLK = 512
TC_GRID = B // TC_BLK


def _ix_body(ekt_ref, x_ref, ix_ref):
    ekt = ekt_ref[...]
    n = jnp.sqrt(jnp.sum(ekt * ekt, axis=0, keepdims=True))
    nkt = ekt / jnp.clip(n, 1e-12)
    x = x_ref[...]
    xs = jnp.sqrt(jnp.sum(x * x, axis=1, keepdims=True))
    q = x / jnp.clip(xs, 1e-12)
    cos = lax.dot_general(q, nkt, (((1,), (0,)), ((), ())),
                          preferred_element_type=jnp.float32)
    m = jnp.max(cos, axis=1, keepdims=True)
    iota = lax.broadcasted_iota(jnp.int32, cos.shape, 1)
    cand = jnp.where(cos >= m, iota, P)
    ix_ref[0, 0, :] = jnp.min(cand, axis=1)


def _topk_indices(e_kt, x_querry):
    return pl.pallas_call(
        _ix_body,
        grid=(TC_GRID,),
        in_specs=[
            pl.BlockSpec((D, P), lambda i: (0, 0)),
            pl.BlockSpec((TC_BLK, D), lambda i: (i, 0)),
        ],
        out_specs=pl.BlockSpec((1, 1, TC_BLK), lambda i: (i, 0, 0)),
        out_shape=jax.ShapeDtypeStruct((TC_GRID, 1, TC_BLK), jnp.int32),
    )(e_kt, x_querry)


def _ek_body(ix_ref, tab_ref, out_ref):
    ixb = ix_ref[0, 0, :]
    iota = lax.broadcasted_iota(jnp.int32, (TC_BLK, P), 1)
    onehot = (iota == ixb[:, None]).astype(jnp.float32)
    out_ref[...] = lax.dot_general(
        onehot, tab_ref[...], (((1,), (0,)), ((), ())),
        preferred_element_type=jnp.float32)


def _ek_gather_tc(ix3, tab):
    return pl.pallas_call(
        _ek_body,
        grid=(TC_GRID,),
        in_specs=[
            pl.BlockSpec((1, 1, TC_BLK), lambda i: (i, 0, 0)),
            pl.BlockSpec((P, DK), lambda i: (0, 0)),
        ],
        out_specs=pl.BlockSpec((TC_BLK, DK), lambda i: (i, 0)),
        out_shape=jax.ShapeDtypeStruct((B, DK), jnp.float32),
    )(ix3, tab)


NC, NS = 2, 16
NW = NC * NS
BPW = B // NW
CHUNK = 16
NCHUNK = BPW // CHUNK


def _gather_body(epv_hbm, idx_hbm, ev_hbm, idx_v, bufa, bufb, sem):
    wid = lax.axis_index("s") * NC + lax.axis_index("c")
    pltpu.sync_copy(idx_hbm.at[wid], idx_v)
    base = wid * BPW

    def chunk(g, carry):
        ca = pltpu.async_copy(epv_hbm.at[idx_v.at[2 * g]], bufa, sem)
        cb = pltpu.async_copy(epv_hbm.at[idx_v.at[2 * g + 1]], bufb, sem)
        ca.wait()
        cb.wait()
        row = base + 2 * g * CHUNK
        pltpu.sync_copy(bufa, ev_hbm.at[pl.ds(row, CHUNK)])
        pltpu.sync_copy(bufb, ev_hbm.at[pl.ds(row + CHUNK, CHUNK)])
        return carry

    lax.fori_loop(0, NCHUNK // 2, chunk, 0)


@functools.cache
def _sc_gather():
    return pl.kernel(
        _gather_body,
        mesh=plsc.VectorSubcoreMesh(core_axis_name="c", subcore_axis_name="s"),
        out_type=jax.ShapeDtypeStruct((B, DK), jnp.float32),
        scratch_types=[
            pltpu.VMEM((NCHUNK, CHUNK), jnp.int32),
            pltpu.VMEM((CHUNK, DK), jnp.float32),
            pltpu.VMEM((CHUNK, DK), jnp.float32),
            pltpu.SemaphoreType.DMA,
        ],
    )


def kernel(x_querry, l, x_block, e_k, e_p):
    ix3 = _topk_indices(e_k.T, x_querry)
    ix = ix3.reshape(B)
    epk = e_p[:, : L // 2, :].reshape(P, DK)
    epv = e_p[:, L // 2:, :].reshape(P, DK)
    ev = _sc_gather()(epv, ix.reshape(NW, NCHUNK, CHUNK))
    ek = _ek_gather_tc(ix3, epk)
    return (ek.reshape(B, L // 2, D), ev.reshape(B, L // 2, D), x_block)


# speedup vs baseline: 1.0311x; 1.0311x over previous
import functools

import jax
import jax.numpy as jnp
from jax import lax
from jax.experimental import pallas as pl
from jax.experimental.pallas import tpu as pltpu
from jax.experimental.pallas import tpu_sc as plsc

B, D, P, L = 16384, 768, 100, 8
DK = (L // 2) * D

TC_BLK = 2048
TC_GRID = B // TC_BLK
EK_BLK = 1024
EK_GRID = B // EK_BLK


def _ix_body(ekt_ref, x_ref, ix_ref):
    ekt = ekt_ref[...]
    n = jnp.sqrt(jnp.sum(ekt * ekt, axis=0, keepdims=True))
    nkt = ekt / jnp.clip(n, 1e-12)
    x = x_ref[...]
    xs = jnp.sqrt(jnp.sum(x * x, axis=1, keepdims=True))
    q = x / jnp.clip(xs, 1e-12)
    cos = lax.dot_general(q, nkt, (((1,), (0,)), ((), ())),
                          preferred_element_type=jnp.float32)
    m = jnp.max(cos, axis=1, keepdims=True)
    iota = lax.broadcasted_iota(jnp.int32, cos.shape, 1)
    cand = jnp.where(cos >= m, iota, P)
    ix_ref[0, 0, :] = jnp.min(cand, axis=1)


def _topk_indices(e_kt, x_querry):
    return pl.pallas_call(
        _ix_body,
        grid=(TC_GRID,),
        in_specs=[
            pl.BlockSpec((D, P), lambda i: (0, 0)),
            pl.BlockSpec((TC_BLK, D), lambda i: (i, 0)),
        ],
        out_specs=pl.BlockSpec((1, 1, TC_BLK), lambda i: (i, 0, 0)),
        out_shape=jax.ShapeDtypeStruct((TC_GRID, 1, TC_BLK), jnp.int32),
    )(e_kt, x_querry)


def _ek_body(ix_ref, tab_ref, out_ref):
    ixb = ix_ref[0, 0, :]
    iota = lax.broadcasted_iota(jnp.int32, (EK_BLK, P), 1)
    onehot = (iota == ixb[:, None]).astype(jnp.float32)
    out_ref[...] = lax.dot_general(
        onehot, tab_ref[...], (((1,), (0,)), ((), ())),
        preferred_element_type=jnp.float32)


def _ek_gather_tc(ix3, tab):
    return pl.pallas_call(
        _ek_body,
        grid=(EK_GRID,),
        in_specs=[
            pl.BlockSpec((1, 1, EK_BLK), lambda i: (i, 0, 0)),
            pl.BlockSpec((P, DK), lambda i: (0, 0)),
        ],
        out_specs=pl.BlockSpec((EK_BLK, DK), lambda i: (i, 0)),
        out_shape=jax.ShapeDtypeStruct((B, DK), jnp.float32),
    )(ix3, tab)


NC, NS = 2, 16
NW = NC * NS
BPW = B // NW
CHUNK = 16
NCHUNK = BPW // CHUNK


def _gather_body(epv_hbm, idx_hbm, ev_hbm, idx_v, bufa, bufb, sem):
    wid = lax.axis_index("s") * NC + lax.axis_index("c")
    pltpu.sync_copy(idx_hbm.at[wid], idx_v)
    base = wid * BPW

    def chunk(g, carry):
        ca = pltpu.async_copy(epv_hbm.at[idx_v.at[2 * g]], bufa, sem)
        cb = pltpu.async_copy(epv_hbm.at[idx_v.at[2 * g + 1]], bufb, sem)
        ca.wait()
        cb.wait()
        row = base + 2 * g * CHUNK
        pltpu.sync_copy(bufa, ev_hbm.at[pl.ds(row, CHUNK)])
        pltpu.sync_copy(bufb, ev_hbm.at[pl.ds(row + CHUNK, CHUNK)])
        return carry

    lax.fori_loop(0, NCHUNK // 2, chunk, 0)


@functools.cache
def _sc_gather():
    return pl.kernel(
        _gather_body,
        mesh=plsc.VectorSubcoreMesh(core_axis_name="c", subcore_axis_name="s"),
        out_type=jax.ShapeDtypeStruct((B, DK), jnp.float32),
        scratch_types=[
            pltpu.VMEM((NCHUNK, CHUNK), jnp.int32),
            pltpu.VMEM((CHUNK, DK), jnp.float32),
            pltpu.VMEM((CHUNK, DK), jnp.float32),
            pltpu.SemaphoreType.DMA,
        ],
    )


def kernel(x_querry, l, x_block, e_k, e_p):
    ix3 = _topk_indices(e_k.T, x_querry)
    ix = ix3.reshape(B)
    epk = e_p[:, : L // 2, :].reshape(P, DK)
    epv = e_p[:, L // 2:, :].reshape(P, DK)
    ev = _sc_gather()(epv, ix.reshape(NW, NCHUNK, CHUNK))
    ek = _ek_gather_tc(ix.reshape(EK_GRID, 1, EK_BLK), epk)
    return (ek.reshape(B, L // 2, D), ev.reshape(B, L // 2, D), x_block)
